# baseline (speedup 1.0000x reference)
import functools

import jax
import jax.numpy as jnp
from jax import lax
from jax.experimental import pallas as pl
from jax.experimental.pallas import tpu as pltpu
from jax.experimental.pallas import tpu_sc as plsc

_EPS = 1e-10
_INV2S = 1.0 / 40.0

_N = 10000
_E = 320000
_NG = 128
_H = 64
_LB = 64
_NBLK = _E // _LB
_NW = 32
_BPW = _NBLK // _NW
_REM = _NBLK - _BPW * _NW
_RING = 3
_NTRI = _BPW // _RING

_SC_PARAMS = pltpu.CompilerParams(needs_layout_passes=False,
                                  use_tc_tiling_on_sc=False)


def _mesh():
    return plsc.VectorSubcoreMesh(core_axis_name="c", subcore_axis_name="s")


def _wid_and_blocks():
    c = lax.axis_index("c")
    s = lax.axis_index("s")
    wid = s * 2 + c
    base = wid * _BPW + jnp.minimum(wid, _REM)
    return c, s, wid, base


def _zero_vmem(ref, rows, cols):
    z = jnp.zeros((16,), ref.dtype)
    if rows == 1:
        def body(i, _):
            ref[pl.ds(i * 16, 16)] = z
            return 0
        lax.fori_loop(0, cols // 16, body, 0)
    else:
        def body(i, _):
            r = i // (cols // 16)
            cc = i % (cols // 16)
            ref[r, pl.ds(cc * 16, 16)] = z
            return 0
        lax.fori_loop(0, rows * (cols // 16), body, 0)


_NFULL = _N // _LB
_NTAIL = _N - _NFULL * _LB


def _zero_shared(shared, zbuf, s):
    for t in range((_NFULL + 15) // 16):
        ci = s + 16 * t

        @pl.when(ci < _NFULL)
        def _():
            pltpu.sync_copy(zbuf, shared.at[pl.ds(ci * _LB, _LB)])

    @pl.when(s == _NFULL % 16)
    def _():
        pltpu.sync_copy(zbuf.at[pl.ds(0, _NTAIL)],
                        shared.at[pl.ds(_NFULL * _LB, _NTAIL)])


def _dump_shared(shared, out_rows, s):
    for t in range((_NFULL + 15) // 16):
        ci = s + 16 * t

        @pl.when(ci < _NFULL)
        def _():
            r0 = ci * _LB
            pltpu.sync_copy(shared.at[pl.ds(r0, _LB)],
                            out_rows.at[pl.ds(r0, _LB)])

    @pl.when(s == _NFULL % 16)
    def _():
        r0 = _NFULL * _LB
        pltpu.sync_copy(shared.at[pl.ds(r0, _NTAIL)],
                        out_rows.at[pl.ds(r0, _NTAIL)])


def _load_worker_blocks(src, dst_all, base, wid, zero_pad):
    if zero_pad:
        z = jnp.zeros((16,), dst_all.dtype)
        for r in range(_BPW, _BPW + _RING - 1):
            for cc in range(_LB // 16):
                dst_all[r, pl.ds(cc * 16, 16)] = z
    pltpu.sync_copy(src.at[pl.ds(base, _BPW)], dst_all.at[pl.ds(0, _BPW)])

    @pl.when(wid < _REM)
    def _():
        pltpu.sync_copy(src.at[pl.ds(base + _BPW, 1)],
                        dst_all.at[pl.ds(_BPW, 1)])


def _decode_idx(pk_all, b, sref, dref):
    for l in range(_LB // 16):
        sl = pl.ds(l * 16, 16)
        pk = pk_all[b, sl]
        sref[sl] = pk & 0xFFFF
        dref[sl] = pk >> 16


def _build_pre():
    out_type = [
        jax.ShapeDtypeStruct((_NW, 1, _N), jnp.int32),
        jax.ShapeDtypeStruct((_NW, 1, _NG), jnp.float32),
    ]
    scratch = [
        pltpu.VMEM((_BPW + _RING - 1, _LB), jnp.int32),
        pltpu.VMEM((_BPW + _RING - 1, _LB), jnp.float32),
        pltpu.VMEM((_N,), jnp.int32),
        pltpu.VMEM((_N,), jnp.int32),
        pltpu.VMEM((16, _NG), jnp.float32),
        pltpu.VMEM((_NG,), jnp.float32),
    ]

    @functools.partial(pl.kernel, out_type=out_type, mesh=_mesh(),
                       scratch_types=scratch, compiler_params=_SC_PARAMS)
    def pre(pki, batch, eimp2, cnt_part, mx_part,
            pk_all, ei_all, hist, batchv, mxtab, mxout):
        c, s, wid, base = _wid_and_blocks()
        nblk = _BPW + jnp.where(wid < _REM, 1, 0)
        lanes = lax.iota(jnp.int32, 16)
        ones_i = jnp.ones((16,), jnp.int32)

        _load_worker_blocks(pki, pk_all, base, wid, False)
        _load_worker_blocks(eimp2, ei_all, base, wid, False)
        pltpu.sync_copy(batch, batchv)
        _zero_vmem(hist, 1, _N)
        _zero_vmem(mxtab, 16, _NG)

        def blk(b, _):
            for l in range(_LB // 16):
                sl = pl.ds(l * 16, 16)
                pk = pk_all[b, sl]
                sv = pk & 0xFFFF
                dv = pk >> 16
                e16 = ei_all[b, sl]
                plsc.addupdate_scatter(hist, [sv], ones_i)
                plsc.addupdate_scatter(hist, [dv], ones_i)
                gb = plsc.load_gather(batchv, [sv])
                cur = plsc.load_gather(mxtab, [lanes, gb])
                plsc.store_scatter(mxtab, [lanes, gb], jnp.maximum(cur, e16))
            return 0

        lax.fori_loop(0, nblk, blk, 0)

        for c0 in range(_NG // 16):
            m = mxtab[0, pl.ds(c0 * 16, 16)]
            for j in range(1, 16):
                m = jnp.maximum(m, mxtab[j, pl.ds(c0 * 16, 16)])
            mxout[pl.ds(c0 * 16, 16)] = m
        pltpu.sync_copy(hist, cnt_part.at[wid, 0])
        pltpu.sync_copy(mxout, mx_part.at[wid, 0])

    return pre


def _build_pass_a():
    out_type = [
        jax.ShapeDtypeStruct((_E, _H), jnp.float32),
        jax.ShapeDtypeStruct((2, _N, _H), jnp.float32),
        jax.ShapeDtypeStruct((_E, 2 * _H), jnp.float32),
    ]
    scratch = [
        pltpu.VMEM((_BPW + _RING - 1, _LB), jnp.int32),
        pltpu.VMEM((_BPW + _RING - 1, _LB), jnp.float32),
        pltpu.VMEM((_N,), jnp.int32),
        pltpu.VMEM((_NG,), jnp.float32),
        pltpu.VMEM((_LB,), jnp.float32),
        [pltpu.VMEM((_LB,), jnp.int32) for _ in range(_RING)],
        [pltpu.VMEM((_LB,), jnp.int32) for _ in range(_RING)],
        [pltpu.VMEM((_LB, _H), jnp.float32) for _ in range(_RING)],
        [pltpu.VMEM((_LB, _H), jnp.float32) for _ in range(_RING)],
        pltpu.VMEM_SHARED((_N, _H), jnp.float32),
        [pltpu.SemaphoreType.DMA for _ in range(_RING)],
        [pltpu.SemaphoreType.DMA for _ in range(_RING)],
        [pltpu.SemaphoreType.DMA for _ in range(_RING)],
        [pltpu.SemaphoreType.DMA for _ in range(_RING)],
    ]

    @functools.partial(pl.kernel, out_type=out_type, mesh=_mesh(),
                       scratch_types=scratch, compiler_params=_SC_PARAMS)
    def pass_a(pki, eimp2, mx, xw0, batch, xw1, agg1p, out_alloc,
               pk_all, ei_all, batchv, mxv, impb,
               sidx, didx, gs, gd, agg_sh, semg_s, semg_d, semwl, semwa):
        del out_alloc
        c, s, wid, base = _wid_and_blocks()

        _load_worker_blocks(pki, pk_all, base, wid, True)
        _load_worker_blocks(eimp2, ei_all, base, wid, False)
        pltpu.sync_copy(batch, batchv)
        pltpu.sync_copy(mx, mxv)
        _zero_vmem(gs[0], _LB, _H)
        _zero_shared(agg_sh, gs[0], s)
        plsc.subcore_barrier()

        def issue_gathers(b, k):
            _decode_idx(pk_all, b, sidx[k], didx[k])
            pltpu.async_copy(xw0.at[sidx[k]], gs[k], semg_s[k])
            pltpu.async_copy(xw0.at[didx[k]], gd[k], semg_d[k])

        def wait_gathers(k):
            pltpu.make_async_copy(xw0.at[sidx[k]], gs[k], semg_s[k]).wait()
            pltpu.make_async_copy(xw0.at[didx[k]], gd[k], semg_d[k]).wait()

        def issue_writes(b, k, sync):
            bi = base + b
            if sync:
                pltpu.sync_copy(gs[k], xw1.at[pl.ds(bi * _LB, _LB)])
                pltpu.sync_copy(gs[k], agg_sh.at[sidx[k]], add=True)
                pltpu.sync_copy(gs[k], agg_sh.at[didx[k]], add=True)
            else:
                pltpu.async_copy(gs[k], xw1.at[pl.ds(bi * _LB, _LB)],
                                 semwl[k])
                pltpu.async_copy(gs[k], agg_sh.at[sidx[k]], semwa[k],
                                 add=True)
                pltpu.async_copy(gs[k], agg_sh.at[didx[k]], semwa[k],
                                 add=True)

        def wait_writes(b, k):
            bi = base + b
            pltpu.make_async_copy(gs[k], xw1.at[pl.ds(bi * _LB, _LB)],
                                  semwl[k]).wait()
            pltpu.make_async_copy(gs[k], agg_sh.at[sidx[k]], semwa[k]).wait()
            pltpu.make_async_copy(gs[k], agg_sh.at[didx[k]], semwa[k]).wait()

        def compute(b, k):
            for l in range(_LB // 16):
                sl = pl.ds(l * 16, 16)
                sv = sidx[k][sl]
                gb = plsc.load_gather(batchv, [sv])
                mxg = plsc.load_gather(mxv, [gb])
                e16 = ei_all[b, sl]
                impb[sl] = ((2.0 * (e16 / (mxg + _EPS)) - 1.0) * _INV2S
                            + 1.0) * 0.5

            def row(i2, _):
                for u in range(2):
                    i = i2 * 2 + u
                    isp = plsc.load_gather(
                        impb, [jnp.full((16,), i, jnp.int32)])
                    for cc in range(_H // 16):
                        sl = pl.ds(cc * 16, 16)
                        gs[k][i, sl] = (gs[k][i, sl] + gd[k][i, sl]) * isp
                return 0

            lax.fori_loop(0, _LB // 2, row, 0)

        for j in range(_RING - 1):
            issue_gathers(j, j)

        def outer(g, _):
            for k in range(_RING):
                b = _RING * g + k
                wait_gathers(k)
                compute(b, k)
                issue_writes(b, k, False)
                if k == 0:
                    @pl.when(g > 0)
                    def _():
                        wait_writes(b - 1, _RING - 1)
                else:
                    wait_writes(b - 1, k - 1)
                issue_gathers(b + _RING - 1, (k + _RING - 1) % _RING)
            return 0

        lax.fori_loop(0, _NTRI, outer, 0)
        for j in range(_RING - 1):
            wait_gathers(j)
        wait_writes(_BPW - 1, _RING - 1)

        @pl.when(wid < _REM)
        def _():
            compute(_BPW, 0)
            issue_writes(_BPW, 0, True)

        plsc.subcore_barrier()
        _dump_shared(agg_sh, agg1p.at[c], s)

    return pass_a


def _build_pass_out(col0, with_agg, bps):
    slb = _LB * bps
    spw = _BPW // bps
    ntri = spw // _RING
    nxw = _REM // bps
    out_type = [
        jax.ShapeDtypeStruct((_NW, _NG, _H), jnp.float32),
    ]
    if with_agg:
        out_type.append(jax.ShapeDtypeStruct((2, _N, _H), jnp.float32))
    scratch = [
        pltpu.VMEM((_BPW + bps * _RING, _LB), jnp.int32),
        pltpu.VMEM((_N,), jnp.int32),
        pltpu.VMEM((_H,), jnp.float32),
        pltpu.VMEM((slb,), jnp.float32),
        pltpu.VMEM((slb,), jnp.int32),
        [pltpu.VMEM((slb,), jnp.int32) for _ in range(_RING)],
        [pltpu.VMEM((slb,), jnp.int32) for _ in range(_RING)],
        [pltpu.VMEM((slb, _H), jnp.float32) for _ in range(_RING)],
        [pltpu.VMEM((slb, _H), jnp.float32) for _ in range(_RING)],
        [pltpu.VMEM((slb, _H), jnp.float32) for _ in range(_RING)],
        pltpu.VMEM((_NG, _H), jnp.float32),
        [pltpu.SemaphoreType.DMA for _ in range(_RING)],
        [pltpu.SemaphoreType.DMA for _ in range(_RING)],
        [pltpu.SemaphoreType.DMA for _ in range(_RING)],
        [pltpu.SemaphoreType.DMA for _ in range(_RING)],
        [pltpu.SemaphoreType.DMA for _ in range(_RING)],
    ]
    if with_agg:
        scratch.append(pltpu.VMEM_SHARED((_N, _H), jnp.float32))

    @functools.partial(pl.kernel, out_type=out_type, mesh=_mesh(),
                       scratch_types=scratch, compiler_params=_SC_PARAMS)
    def pass_out(pki, xw, agg, packed, bias, out_ref, gpart, *rest):
        if with_agg:
            (aggnp, pk_all, pkv, bv, invdb, gbb, sidx, didx,
             xb, gs, gd, gtab, semx, semg_s, semg_d, semwl, semwa,
             agg_sh) = rest
        else:
            (pk_all, pkv, bv, invdb, gbb, sidx, didx,
             xb, gs, gd, gtab, semx, semg_s, semg_d, semwl, semwa) = rest
        c, s, wid, _ = _wid_and_blocks()
        base = wid * _BPW + bps * jnp.minimum(wid, nxw)
        lanes = lax.iota(jnp.int32, 16)

        zi = jnp.zeros((16,), jnp.int32)
        for r in range(_BPW, _BPW + bps * _RING):
            for cc in range(_LB // 16):
                pk_all[r, pl.ds(cc * 16, 16)] = zi
        pltpu.sync_copy(pki.at[pl.ds(base, _BPW)],
                        pk_all.at[pl.ds(0, _BPW)])

        @pl.when(wid < nxw)
        def _():
            pltpu.sync_copy(pki.at[pl.ds(base + _BPW, bps)],
                            pk_all.at[pl.ds(_BPW, bps)])

        pltpu.sync_copy(packed, pkv)
        pltpu.sync_copy(bias, bv)
        _zero_vmem(gtab, _NG, _H)
        if with_agg:
            _zero_vmem(gs[0], slb, _H)
            _zero_shared(agg_sh, gs[0], s)
            plsc.subcore_barrier()
        bc = [bv[pl.ds(cc * 16, 16)] for cc in range(_H // 16)]

        def eoff(b):
            return (jnp.minimum(base + b * bps, _NBLK - bps)) * _LB

        def issue_rowgathers(b, k):
            for r in range(bps):
                for l in range(_LB // 16):
                    sl = pl.ds(r * _LB + l * 16, 16)
                    pk = pk_all[b * bps + r, pl.ds(l * 16, 16)]
                    sidx[k][sl] = pk & 0xFFFF
                    didx[k][sl] = pk >> 16
            pltpu.async_copy(agg.at[sidx[k]], gs[k], semg_s[k])
            pltpu.async_copy(agg.at[didx[k]], gd[k], semg_d[k])
            pltpu.async_copy(xw.at[pl.ds(eoff(b), slb)], xb[k], semx[k])

        def wait_rowgathers(k):
            pltpu.make_async_copy(agg.at[sidx[k]], gs[k], semg_s[k]).wait()
            pltpu.make_async_copy(agg.at[didx[k]], gd[k], semg_d[k]).wait()
            pltpu.make_async_copy(xw.at[pl.ds(0, slb)], xb[k],
                                  semx[k]).wait()

        def issue_writes(b, k, sync):
            dst = out_ref.at[pl.ds(eoff(b), slb), pl.ds(col0, _H)]
            if sync:
                pltpu.sync_copy(xb[k], dst)
                if with_agg:
                    pltpu.sync_copy(xb[k], agg_sh.at[sidx[k]], add=True)
                    pltpu.sync_copy(xb[k], agg_sh.at[didx[k]], add=True)
            else:
                pltpu.async_copy(xb[k], dst, semwl[k])
                if with_agg:
                    pltpu.async_copy(xb[k], agg_sh.at[sidx[k]], semwa[k],
                                     add=True)
                    pltpu.async_copy(xb[k], agg_sh.at[didx[k]], semwa[k],
                                     add=True)

        def wait_writes(b, k):
            pltpu.make_async_copy(
                xb[k], out_ref.at[pl.ds(eoff(b), slb), pl.ds(col0, _H)],
                semwl[k]).wait()
            if with_agg:
                pltpu.make_async_copy(xb[k], agg_sh.at[sidx[k]],
                                      semwa[k]).wait()
                pltpu.make_async_copy(xb[k], agg_sh.at[didx[k]],
                                      semwa[k]).wait()

        def compute(b, k):
            for l in range(slb // 16):
                sl = pl.ds(l * 16, 16)
                pks = plsc.load_gather(pkv, [sidx[k][sl]])
                pkd = plsc.load_gather(pkv, [didx[k][sl]])
                ms = (pks >> 16).astype(jnp.float32)
                md = (pkd >> 16).astype(jnp.float32)
                invdb[sl] = 1.0 / (1.0 + ms + md)
                gbb[sl] = pks & 0xFFFF

            def row(i2, _):
                for u in range(2):
                    i = i2 * 2 + u
                    full_i = jnp.full((16,), i, jnp.int32)
                    dsp = plsc.load_gather(invdb, [full_i])
                    gidx = plsc.load_gather(gbb, [full_i])
                    for cc in range(_H // 16):
                        sl = pl.ds(cc * 16, 16)
                        h = (gs[k][i, sl] + gd[k][i, sl]
                             + xb[k][i, sl]) * dsp
                        h = jnp.maximum(h + bc[cc], 0.0)
                        xb[k][i, sl] = h
                        plsc.addupdate_scatter(
                            gtab, [gidx, lanes + cc * 16], h)
                return 0

            lax.fori_loop(0, slb // 2, row, 0)

        for j in range(_RING - 1):
            issue_rowgathers(j, j)

        def outer(g, _):
            for k in range(_RING):
                b = _RING * g + k
                wait_rowgathers(k)
                compute(b, k)
                issue_writes(b, k, False)
                if k == 0:
                    @pl.when(g > 0)
                    def _():
                        wait_writes(b - 1, _RING - 1)
                else:
                    wait_writes(b - 1, k - 1)
                issue_rowgathers(b + _RING - 1, (k + _RING - 1) % _RING)
            return 0

        lax.fori_loop(0, ntri, outer, 0)
        for j in range(_RING - 1):
            wait_rowgathers(j)
        wait_writes(spw - 1, _RING - 1)

        @pl.when(wid < nxw)
        def _():
            issue_rowgathers(spw, 0)
            wait_rowgathers(0)
            compute(spw, 0)
            issue_writes(spw, 0, True)

        pltpu.sync_copy(gtab, gpart.at[wid])
        if with_agg:
            plsc.subcore_barrier()
            _dump_shared(agg_sh, aggnp.at[c], s)

    return pass_out


def _tc_prep(x, w0, cnt_part, mx_part, batch):
    def body(x_r, w0_r, cp_r, mp_r, b_r, xw0_r, invc_r, pk_r, mx_r):
        xw0_r[...] = jnp.dot(x_r[...], w0_r[...],
                             preferred_element_type=jnp.float32)
        cnt = jnp.sum(cp_r[...], axis=0)
        m = cnt >= 2
        cf = cnt.astype(jnp.float32)
        invc_r[...] = jnp.where(m, 1.0 / jnp.maximum(cf, 1.0), 0.0)
        pk_r[...] = jnp.where(m, 1 << 16, 0) + b_r[...]
        mx_r[...] = jnp.max(mp_r[...], axis=0)

    return pl.pallas_call(
        body,
        out_shape=[
            jax.ShapeDtypeStruct((_N, _H), jnp.float32),
            jax.ShapeDtypeStruct((_N,), jnp.float32),
            jax.ShapeDtypeStruct((_N,), jnp.int32),
            jax.ShapeDtypeStruct((_NG,), jnp.float32),
        ],
    )(x, w0, cnt_part, mx_part, batch)


def _tc_combine(p2, invc, w=None):
    def body(p_r, i_r, *rest):
        a = (p_r[0] + p_r[1]) * i_r[...][:, None]
        if w is None:
            rest[-1][...] = a
        else:
            rest[-1][...] = jnp.dot(a, rest[0][...],
                                    preferred_element_type=jnp.float32)

    args = (p2, invc) if w is None else (p2, invc, w)
    return pl.pallas_call(
        body,
        out_shape=jax.ShapeDtypeStruct((_N, _H), jnp.float32),
    )(*args)


def _tc_edge_matmul(h_src, w1, col0):
    bs = 1000

    def body(h_r, w_r, o_r):
        o_r[...] = jnp.dot(h_r[...][:, col0:col0 + _H], w_r[...],
                           preferred_element_type=jnp.float32)

    return pl.pallas_call(
        body,
        grid=(_E // bs,),
        in_specs=[
            pl.BlockSpec((bs, 2 * _H), lambda i: (i, 0)),
            pl.BlockSpec((_H, _H), lambda i: (0, 0)),
        ],
        out_specs=pl.BlockSpec((bs, _H), lambda i: (i, 0)),
        out_shape=jax.ShapeDtypeStruct((_E, _H), jnp.float32),
    )(h_src, w1)


def _tc_final(gp1, gp2):
    def body(a_r, b_r, o_r):
        o_r[...] = jnp.sum(a_r[...], axis=0) + jnp.sum(b_r[...], axis=0)

    return pl.pallas_call(
        body,
        out_shape=jax.ShapeDtypeStruct((_NG, _H), jnp.float32),
    )(gp1, gp2)


def kernel(x, edge_index, edge_attr, batch, edge_imp, W0, b0, W1, b1):
    del edge_attr
    edge_index = edge_index.astype(jnp.int32)
    batch = batch.astype(jnp.int32)
    pki = (edge_index[0] + (edge_index[1] << 16)).reshape(_NBLK, _LB)
    eimp2 = edge_imp.reshape(_NBLK, _LB)

    cnt_part, mx_part = _build_pre()(pki, batch, eimp2)
    xw0, invc, packed, mx = _tc_prep(
        x, W0, cnt_part.reshape(_NW, _N), mx_part.reshape(_NW, _NG), batch)

    xw1, agg1p, out_alloc = _build_pass_a()(pki, eimp2, mx, xw0, batch)
    agg1 = _tc_combine(agg1p, invc)

    out_ref = jax.new_ref(out_alloc)
    gpart1, aggh1p = _build_pass_out(0, True, 1)(
        pki, xw1, agg1, packed, b0, out_ref)

    xw2 = _tc_edge_matmul(out_ref[...], W1, 0)
    agg2 = _tc_combine(aggh1p, invc, W1)

    gpart2, = _build_pass_out(_H, False, 2)(
        pki, xw2, agg2, packed, b1, out_ref)

    graph_emb = _tc_final(gpart1, gpart2)
    return graph_emb, out_ref[...]


# speedup vs baseline: 1.0758x; 1.0758x over previous
import functools

import jax
import jax.numpy as jnp
from jax import lax
from jax.experimental import pallas as pl
from jax.experimental.pallas import tpu as pltpu
from jax.experimental.pallas import tpu_sc as plsc

_EPS = 1e-10
_INV2S = 1.0 / 40.0

_N = 10000
_E = 320000
_NG = 128
_H = 64
_LB = 64
_NBLK = _E // _LB
_NW = 32
_BPW = _NBLK // _NW
_REM = _NBLK - _BPW * _NW
_RING = 3
_NTRI = _BPW // _RING

_SC_PARAMS = pltpu.CompilerParams(needs_layout_passes=False,
                                  use_tc_tiling_on_sc=False)


def _mesh():
    return plsc.VectorSubcoreMesh(core_axis_name="c", subcore_axis_name="s")


def _wid_and_blocks():
    c = lax.axis_index("c")
    s = lax.axis_index("s")
    wid = s * 2 + c
    base = wid * _BPW + jnp.minimum(wid, _REM)
    return c, s, wid, base


def _zero_vmem(ref, rows, cols):
    z = jnp.zeros((16,), ref.dtype)
    if rows == 1:
        def body(i, _):
            ref[pl.ds(i * 16, 16)] = z
            return 0
        lax.fori_loop(0, cols // 16, body, 0)
    else:
        def body(i, _):
            r = i // (cols // 16)
            cc = i % (cols // 16)
            ref[r, pl.ds(cc * 16, 16)] = z
            return 0
        lax.fori_loop(0, rows * (cols // 16), body, 0)


_NFULL = _N // _LB
_NTAIL = _N - _NFULL * _LB


def _zero_shared(shared, zbuf, s):
    for t in range((_NFULL + 15) // 16):
        ci = s + 16 * t

        @pl.when(ci < _NFULL)
        def _():
            pltpu.sync_copy(zbuf, shared.at[pl.ds(ci * _LB, _LB)])

    @pl.when(s == _NFULL % 16)
    def _():
        pltpu.sync_copy(zbuf.at[pl.ds(0, _NTAIL)],
                        shared.at[pl.ds(_NFULL * _LB, _NTAIL)])


def _dump_shared(shared, out_rows, s):
    for t in range((_NFULL + 15) // 16):
        ci = s + 16 * t

        @pl.when(ci < _NFULL)
        def _():
            r0 = ci * _LB
            pltpu.sync_copy(shared.at[pl.ds(r0, _LB)],
                            out_rows.at[pl.ds(r0, _LB)])

    @pl.when(s == _NFULL % 16)
    def _():
        r0 = _NFULL * _LB
        pltpu.sync_copy(shared.at[pl.ds(r0, _NTAIL)],
                        out_rows.at[pl.ds(r0, _NTAIL)])


def _load_worker_blocks(src, dst_all, base, wid, zero_pad):
    if zero_pad:
        z = jnp.zeros((16,), dst_all.dtype)
        for r in range(_BPW, _BPW + _RING - 1):
            for cc in range(_LB // 16):
                dst_all[r, pl.ds(cc * 16, 16)] = z
    pltpu.sync_copy(src.at[pl.ds(base, _BPW)], dst_all.at[pl.ds(0, _BPW)])

    @pl.when(wid < _REM)
    def _():
        pltpu.sync_copy(src.at[pl.ds(base + _BPW, 1)],
                        dst_all.at[pl.ds(_BPW, 1)])


def _decode_idx(pk_all, b, sref, dref):
    for l in range(_LB // 16):
        sl = pl.ds(l * 16, 16)
        pk = pk_all[b, sl]
        sref[sl] = pk & 0xFFFF
        dref[sl] = pk >> 16


def _build_pre():
    out_type = [
        jax.ShapeDtypeStruct((_NW, 1, _N), jnp.int32),
        jax.ShapeDtypeStruct((_NW, 1, _NG), jnp.float32),
    ]
    scratch = [
        pltpu.VMEM((_BPW + _RING - 1, _LB), jnp.int32),
        pltpu.VMEM((_BPW + _RING - 1, _LB), jnp.float32),
        pltpu.VMEM((_N,), jnp.int32),
        pltpu.VMEM((_N,), jnp.int32),
        pltpu.VMEM((16, _NG), jnp.float32),
        pltpu.VMEM((_NG,), jnp.float32),
    ]

    @functools.partial(pl.kernel, out_type=out_type, mesh=_mesh(),
                       scratch_types=scratch, compiler_params=_SC_PARAMS)
    def pre(pki, batch, eimp2, cnt_part, mx_part,
            pk_all, ei_all, hist, batchv, mxtab, mxout):
        c, s, wid, base = _wid_and_blocks()
        nblk = _BPW + jnp.where(wid < _REM, 1, 0)
        lanes = lax.iota(jnp.int32, 16)
        ones_i = jnp.ones((16,), jnp.int32)

        _load_worker_blocks(pki, pk_all, base, wid, False)
        _load_worker_blocks(eimp2, ei_all, base, wid, False)
        pltpu.sync_copy(batch, batchv)
        _zero_vmem(hist, 1, _N)
        _zero_vmem(mxtab, 16, _NG)

        def blk(b, _):
            for l in range(_LB // 16):
                sl = pl.ds(l * 16, 16)
                pk = pk_all[b, sl]
                sv = pk & 0xFFFF
                dv = pk >> 16
                e16 = ei_all[b, sl]
                plsc.addupdate_scatter(hist, [sv], ones_i)
                plsc.addupdate_scatter(hist, [dv], ones_i)
                gb = plsc.load_gather(batchv, [sv])
                cur = plsc.load_gather(mxtab, [lanes, gb])
                plsc.store_scatter(mxtab, [lanes, gb], jnp.maximum(cur, e16))
            return 0

        lax.fori_loop(0, nblk, blk, 0)

        for c0 in range(_NG // 16):
            m = mxtab[0, pl.ds(c0 * 16, 16)]
            for j in range(1, 16):
                m = jnp.maximum(m, mxtab[j, pl.ds(c0 * 16, 16)])
            mxout[pl.ds(c0 * 16, 16)] = m
        pltpu.sync_copy(hist, cnt_part.at[wid, 0])
        pltpu.sync_copy(mxout, mx_part.at[wid, 0])

    return pre


def _build_pass_a():
    out_type = [
        jax.ShapeDtypeStruct((_E, _H), jnp.float32),
        jax.ShapeDtypeStruct((2, _N, _H), jnp.float32),
        jax.ShapeDtypeStruct((_E, 2 * _H), jnp.float32),
    ]
    scratch = [
        pltpu.VMEM((_BPW + _RING - 1, _LB), jnp.int32),
        pltpu.VMEM((_BPW + _RING - 1, _LB), jnp.float32),
        pltpu.VMEM((_N,), jnp.int32),
        pltpu.VMEM((_NG,), jnp.float32),
        pltpu.VMEM((_LB,), jnp.float32),
        [pltpu.VMEM((_LB,), jnp.int32) for _ in range(_RING)],
        [pltpu.VMEM((_LB,), jnp.int32) for _ in range(_RING)],
        [pltpu.VMEM((_LB, _H), jnp.float32) for _ in range(_RING)],
        [pltpu.VMEM((_LB, _H), jnp.float32) for _ in range(_RING)],
        pltpu.VMEM_SHARED((_N, _H), jnp.float32),
        [pltpu.SemaphoreType.DMA for _ in range(_RING)],
        [pltpu.SemaphoreType.DMA for _ in range(_RING)],
        [pltpu.SemaphoreType.DMA for _ in range(_RING)],
        [pltpu.SemaphoreType.DMA for _ in range(_RING)],
    ]

    @functools.partial(pl.kernel, out_type=out_type, mesh=_mesh(),
                       scratch_types=scratch, compiler_params=_SC_PARAMS)
    def pass_a(pki, eimp2, mx, xw0, batch, xw1, agg1p, out_alloc,
               pk_all, ei_all, batchv, mxv, impb,
               sidx, didx, gs, gd, agg_sh, semg_s, semg_d, semwl, semwa):
        del out_alloc
        c, s, wid, base = _wid_and_blocks()

        _load_worker_blocks(pki, pk_all, base, wid, True)
        _load_worker_blocks(eimp2, ei_all, base, wid, False)
        pltpu.sync_copy(batch, batchv)
        pltpu.sync_copy(mx, mxv)
        _zero_vmem(gs[0], _LB, _H)
        _zero_shared(agg_sh, gs[0], s)
        plsc.subcore_barrier()

        def issue_gathers(b, k):
            _decode_idx(pk_all, b, sidx[k], didx[k])
            pltpu.async_copy(xw0.at[sidx[k]], gs[k], semg_s[k])
            pltpu.async_copy(xw0.at[didx[k]], gd[k], semg_d[k])

        def wait_gathers(k):
            pltpu.make_async_copy(xw0.at[sidx[k]], gs[k], semg_s[k]).wait()
            pltpu.make_async_copy(xw0.at[didx[k]], gd[k], semg_d[k]).wait()

        def issue_writes(b, k, sync):
            bi = base + b
            if sync:
                pltpu.sync_copy(gs[k], xw1.at[pl.ds(bi * _LB, _LB)])
                pltpu.sync_copy(gs[k], agg_sh.at[sidx[k]], add=True)
                pltpu.sync_copy(gs[k], agg_sh.at[didx[k]], add=True)
            else:
                pltpu.async_copy(gs[k], xw1.at[pl.ds(bi * _LB, _LB)],
                                 semwl[k])
                pltpu.async_copy(gs[k], agg_sh.at[sidx[k]], semwa[k],
                                 add=True)
                pltpu.async_copy(gs[k], agg_sh.at[didx[k]], semwa[k],
                                 add=True)

        def wait_writes(b, k):
            bi = base + b
            pltpu.make_async_copy(gs[k], xw1.at[pl.ds(bi * _LB, _LB)],
                                  semwl[k]).wait()
            pltpu.make_async_copy(gs[k], agg_sh.at[sidx[k]], semwa[k]).wait()
            pltpu.make_async_copy(gs[k], agg_sh.at[didx[k]], semwa[k]).wait()

        def compute(b, k):
            for l in range(_LB // 16):
                sl = pl.ds(l * 16, 16)
                sv = sidx[k][sl]
                gb = plsc.load_gather(batchv, [sv])
                mxg = plsc.load_gather(mxv, [gb])
                e16 = ei_all[b, sl]
                impb[sl] = ((2.0 * (e16 / (mxg + _EPS)) - 1.0) * _INV2S
                            + 1.0) * 0.5

            def row(i2, _):
                for u in range(2):
                    i = i2 * 2 + u
                    isp = plsc.load_gather(
                        impb, [jnp.full((16,), i, jnp.int32)])
                    for cc in range(_H // 16):
                        sl = pl.ds(cc * 16, 16)
                        gs[k][i, sl] = (gs[k][i, sl] + gd[k][i, sl]) * isp
                return 0

            lax.fori_loop(0, _LB // 2, row, 0)

        for j in range(_RING - 1):
            issue_gathers(j, j)

        def outer(g, _):
            for k in range(_RING):
                b = _RING * g + k
                wait_gathers(k)
                compute(b, k)
                issue_writes(b, k, False)
                if k == 0:
                    @pl.when(g > 0)
                    def _():
                        wait_writes(b - 1, _RING - 1)
                else:
                    wait_writes(b - 1, k - 1)
                issue_gathers(b + _RING - 1, (k + _RING - 1) % _RING)
            return 0

        lax.fori_loop(0, _NTRI, outer, 0)
        for j in range(_RING - 1):
            wait_gathers(j)
        wait_writes(_BPW - 1, _RING - 1)

        @pl.when(wid < _REM)
        def _():
            compute(_BPW, 0)
            issue_writes(_BPW, 0, True)

        plsc.subcore_barrier()
        _dump_shared(agg_sh, agg1p.at[c], s)

    return pass_a


def _build_pass_out(col0, with_agg, bps):
    slb = _LB * bps
    spw = _BPW // bps
    ntri = spw // _RING
    nxw = _REM // bps
    out_type = [
        jax.ShapeDtypeStruct((_NW, _NG, _H), jnp.float32),
    ]
    if with_agg:
        out_type.append(jax.ShapeDtypeStruct((2, _N, _H), jnp.float32))
    scratch = [
        pltpu.VMEM((_BPW + bps * _RING, _LB), jnp.int32),
        pltpu.VMEM((_N,), jnp.int32),
        pltpu.VMEM((_H,), jnp.float32),
        pltpu.VMEM((slb,), jnp.float32),
        pltpu.VMEM((slb,), jnp.int32),
        [pltpu.VMEM((slb,), jnp.int32) for _ in range(_RING)],
        [pltpu.VMEM((slb,), jnp.int32) for _ in range(_RING)],
        [pltpu.VMEM((slb, _H), jnp.float32) for _ in range(_RING)],
        [pltpu.VMEM((slb, _H), jnp.float32) for _ in range(_RING)],
        [pltpu.VMEM((slb, _H), jnp.float32) for _ in range(_RING)],
        pltpu.VMEM((_NG, _H), jnp.float32),
        [pltpu.SemaphoreType.DMA for _ in range(_RING)],
        [pltpu.SemaphoreType.DMA for _ in range(_RING)],
        [pltpu.SemaphoreType.DMA for _ in range(_RING)],
        [pltpu.SemaphoreType.DMA for _ in range(_RING)],
        [pltpu.SemaphoreType.DMA for _ in range(_RING)],
    ]
    if with_agg:
        scratch.append(pltpu.VMEM_SHARED((_N, _H), jnp.float32))

    @functools.partial(pl.kernel, out_type=out_type, mesh=_mesh(),
                       scratch_types=scratch, compiler_params=_SC_PARAMS)
    def pass_out(pki, xw, agg, packed, bias, out_ref, gpart, *rest):
        if with_agg:
            (aggnp, pk_all, pkv, bv, invdb, gbb, sidx, didx,
             xb, gs, gd, gtab, semx, semg_s, semg_d, semwl, semwa,
             agg_sh) = rest
        else:
            (pk_all, pkv, bv, invdb, gbb, sidx, didx,
             xb, gs, gd, gtab, semx, semg_s, semg_d, semwl, semwa) = rest
        c, s, wid, _ = _wid_and_blocks()
        base = wid * _BPW + bps * jnp.minimum(wid, nxw)
        lanes = lax.iota(jnp.int32, 16)

        zi = jnp.zeros((16,), jnp.int32)
        for r in range(_BPW, _BPW + bps * _RING):
            for cc in range(_LB // 16):
                pk_all[r, pl.ds(cc * 16, 16)] = zi
        pltpu.sync_copy(pki.at[pl.ds(base, _BPW)],
                        pk_all.at[pl.ds(0, _BPW)])

        @pl.when(wid < nxw)
        def _():
            pltpu.sync_copy(pki.at[pl.ds(base + _BPW, bps)],
                            pk_all.at[pl.ds(_BPW, bps)])

        pltpu.sync_copy(packed, pkv)
        pltpu.sync_copy(bias, bv)
        _zero_vmem(gtab, _NG, _H)
        if with_agg:
            _zero_vmem(gs[0], slb, _H)
            _zero_shared(agg_sh, gs[0], s)
            plsc.subcore_barrier()
        bc = [bv[pl.ds(cc * 16, 16)] for cc in range(_H // 16)]

        def eoff(b):
            return (jnp.minimum(base + b * bps, _NBLK - bps)) * _LB

        def issue_rowgathers(b, k):
            for r in range(bps):
                for l in range(_LB // 16):
                    sl = pl.ds(r * _LB + l * 16, 16)
                    pk = pk_all[b * bps + r, pl.ds(l * 16, 16)]
                    sidx[k][sl] = pk & 0xFFFF
                    didx[k][sl] = pk >> 16
            pltpu.async_copy(agg.at[sidx[k]], gs[k], semg_s[k])
            pltpu.async_copy(agg.at[didx[k]], gd[k], semg_d[k])
            pltpu.async_copy(xw.at[pl.ds(eoff(b), slb)], xb[k], semx[k])

        def wait_rowgathers(k):
            pltpu.make_async_copy(agg.at[sidx[k]], gs[k], semg_s[k]).wait()
            pltpu.make_async_copy(agg.at[didx[k]], gd[k], semg_d[k]).wait()
            pltpu.make_async_copy(xw.at[pl.ds(0, slb)], xb[k],
                                  semx[k]).wait()

        def issue_writes(b, k, sync):
            dst = out_ref.at[pl.ds(eoff(b), slb), pl.ds(col0, _H)]
            if sync:
                pltpu.sync_copy(xb[k], dst)
                if with_agg:
                    pltpu.sync_copy(xb[k], agg_sh.at[sidx[k]], add=True)
                    pltpu.sync_copy(xb[k], agg_sh.at[didx[k]], add=True)
            else:
                pltpu.async_copy(xb[k], dst, semwl[k])
                if with_agg:
                    pltpu.async_copy(xb[k], agg_sh.at[sidx[k]], semwa[k],
                                     add=True)
                    pltpu.async_copy(xb[k], agg_sh.at[didx[k]], semwa[k],
                                     add=True)

        def wait_writes(b, k):
            pltpu.make_async_copy(
                xb[k], out_ref.at[pl.ds(eoff(b), slb), pl.ds(col0, _H)],
                semwl[k]).wait()
            if with_agg:
                pltpu.make_async_copy(xb[k], agg_sh.at[sidx[k]],
                                      semwa[k]).wait()
                pltpu.make_async_copy(xb[k], agg_sh.at[didx[k]],
                                      semwa[k]).wait()

        def compute(b, k):
            for l in range(slb // 16):
                sl = pl.ds(l * 16, 16)
                pks = plsc.load_gather(pkv, [sidx[k][sl]])
                pkd = plsc.load_gather(pkv, [didx[k][sl]])
                ms = (pks >> 16).astype(jnp.float32)
                md = (pkd >> 16).astype(jnp.float32)
                invdb[sl] = 1.0 / (1.0 + ms + md)
                gbb[sl] = pks & 0xFFFF

            def row(i2, _):
                for u in range(2):
                    i = i2 * 2 + u
                    full_i = jnp.full((16,), i, jnp.int32)
                    dsp = plsc.load_gather(invdb, [full_i])
                    gidx = plsc.load_gather(gbb, [full_i])
                    for cc in range(_H // 16):
                        sl = pl.ds(cc * 16, 16)
                        h = (gs[k][i, sl] + gd[k][i, sl]
                             + xb[k][i, sl]) * dsp
                        h = jnp.maximum(h + bc[cc], 0.0)
                        xb[k][i, sl] = h
                        plsc.addupdate_scatter(
                            gtab, [gidx, lanes + cc * 16], h)
                return 0

            lax.fori_loop(0, slb // 2, row, 0)

        for j in range(_RING - 1):
            issue_rowgathers(j, j)

        def outer(g, _):
            for k in range(_RING):
                b = _RING * g + k
                wait_rowgathers(k)
                compute(b, k)
                issue_writes(b, k, False)
                if k == 0:
                    @pl.when(g > 0)
                    def _():
                        wait_writes(b - 1, _RING - 1)
                else:
                    wait_writes(b - 1, k - 1)
                issue_rowgathers(b + _RING - 1, (k + _RING - 1) % _RING)
            return 0

        lax.fori_loop(0, ntri, outer, 0)
        for j in range(_RING - 1):
            wait_rowgathers(j)
        wait_writes(spw - 1, _RING - 1)

        @pl.when(wid < nxw)
        def _():
            issue_rowgathers(spw, 0)
            wait_rowgathers(0)
            compute(spw, 0)
            issue_writes(spw, 0, True)

        pltpu.sync_copy(gtab, gpart.at[wid])
        if with_agg:
            plsc.subcore_barrier()
            _dump_shared(agg_sh, aggnp.at[c], s)

    return pass_out


def _tc_prep(x, w0, cnt_part, mx_part, batch):
    def body(x_r, w0_r, cp_r, mp_r, b_r, xw0_r, invc_r, pk_r, mx_r):
        xw0_r[...] = jnp.dot(x_r[...], w0_r[...],
                             preferred_element_type=jnp.float32)
        cnt = jnp.sum(cp_r[...], axis=0)
        m = cnt >= 2
        cf = cnt.astype(jnp.float32)
        invc_r[...] = jnp.where(m, 1.0 / jnp.maximum(cf, 1.0), 0.0)
        pk_r[...] = jnp.where(m, 1 << 16, 0) + b_r[...]
        mx_r[...] = jnp.max(mp_r[...], axis=0)

    return pl.pallas_call(
        body,
        out_shape=[
            jax.ShapeDtypeStruct((_N, _H), jnp.float32),
            jax.ShapeDtypeStruct((_N,), jnp.float32),
            jax.ShapeDtypeStruct((_N,), jnp.int32),
            jax.ShapeDtypeStruct((_NG,), jnp.float32),
        ],
    )(x, w0, cnt_part, mx_part, batch)


def _tc_combine(p2, invc, w=None):
    def body(p_r, i_r, *rest):
        a = (p_r[0] + p_r[1]) * i_r[...][:, None]
        if w is None:
            rest[-1][...] = a
        else:
            rest[-1][...] = jnp.dot(a, rest[0][...],
                                    preferred_element_type=jnp.float32)

    args = (p2, invc) if w is None else (p2, invc, w)
    return pl.pallas_call(
        body,
        out_shape=jax.ShapeDtypeStruct((_N, _H), jnp.float32),
    )(*args)


def _tc_edge_matmul(h_src, w1, col0):
    bs = 1000

    def body(h_r, w_r, o_r):
        o_r[...] = jnp.dot(h_r[...][:, col0:col0 + _H], w_r[...],
                           preferred_element_type=jnp.float32)

    return pl.pallas_call(
        body,
        grid=(_E // bs,),
        in_specs=[
            pl.BlockSpec((bs, 2 * _H), lambda i: (i, 0)),
            pl.BlockSpec((_H, _H), lambda i: (0, 0)),
        ],
        out_specs=pl.BlockSpec((bs, _H), lambda i: (i, 0)),
        out_shape=jax.ShapeDtypeStruct((_E, _H), jnp.float32),
    )(h_src, w1)


def _tc_final(gp1, gp2):
    def body(a_r, b_r, o_r):
        o_r[...] = jnp.sum(a_r[...], axis=0) + jnp.sum(b_r[...], axis=0)

    return pl.pallas_call(
        body,
        out_shape=jax.ShapeDtypeStruct((_NG, _H), jnp.float32),
    )(gp1, gp2)


def kernel(x, edge_index, edge_attr, batch, edge_imp, W0, b0, W1, b1):
    del edge_attr
    edge_index = edge_index.astype(jnp.int32)
    batch = batch.astype(jnp.int32)
    pki = (edge_index[0] + (edge_index[1] << 16)).reshape(_NBLK, _LB)
    eimp2 = edge_imp.reshape(_NBLK, _LB)

    cnt_part, mx_part = _build_pre()(pki, batch, eimp2)
    xw0, invc, packed, mx = _tc_prep(
        x, W0, cnt_part.reshape(_NW, _N), mx_part.reshape(_NW, _NG), batch)

    xw1, agg1p, out_alloc = _build_pass_a()(pki, eimp2, mx, xw0, batch)
    agg1 = _tc_combine(agg1p, invc)

    out_ref = jax.new_ref(out_alloc)
    gpart1, aggh1p = _build_pass_out(0, True, 1)(
        pki, xw1, agg1, packed, b0, out_ref)

    xw2 = _tc_edge_matmul(out_ref[...], W1, 0)
    agg2 = _tc_combine(aggh1p, invc, W1)

    gpart2, = _build_pass_out(_H, False, 1)(
        pki, xw2, agg2, packed, b1, out_ref)

    graph_emb = _tc_final(gpart1, gpart2)
    return graph_emb, out_ref[...]
